# baseline (speedup 1.0000x reference)
import functools

import jax
import jax.numpy as jnp
from jax import lax
from jax.experimental import pallas as pl
from jax.experimental.pallas import tpu as pltpu
from jax.experimental.pallas import tpu_sc as plsc

K = 1024
DIM = 64
SEQ = 576
ROWS = 64 * SEQ
NBLK = 64

NW = 32
RPW = ROWS // NW
CHUNK = 128
NCH = RPW // CHUNK
IDX_ROWS = ROWS // CHUNK


def _dist_argmin(zt_ref, cb2_ref, c2b_ref, z2_ref, idx_ref, loss_ref):
    i = pl.program_id(0)
    ztb = zt_ref[...]
    logits2 = lax.dot_general(
        cb2_ref[...], ztb, (((1,), (0,)), ((), ())),
        preferred_element_type=jnp.float32)
    score = (c2b_ref[...] + z2_ref[0]) - logits2
    colmin = jnp.min(score, axis=0, keepdims=True)
    iota_f = lax.broadcasted_iota(
        jnp.int32, score.shape, 0).astype(jnp.float32)
    idx_f = jnp.min(jnp.where(score == colmin, iota_f, jnp.float32(K)),
                    axis=0)
    idx_ref[0, 0, :] = idx_f.astype(jnp.int32)
    loss_part = jnp.sum(colmin).reshape(1, 1)

    @pl.when(i == 0)
    def _init():
        loss_ref[...] = loss_part

    @pl.when(i > 0)
    def _acc():
        loss_ref[...] += loss_part


def _sc_gather_hist(cb_hbm, idx_hbm, zeros_hbm, zq_hbm, cnt_hbm,
                    idx_v, rows_v, hist_v, cnt_v, sem):
    wid = lax.axis_index("s") * 2 + lax.axis_index("c")

    pltpu.sync_copy(idx_hbm.at[wid], idx_v)
    copies = [
        pltpu.async_copy(cb_hbm.at[idx_v.at[j]],
                         rows_v.at[pl.ds(j * CHUNK, CHUNK)], sem)
        for j in range(NCH)
    ]

    pltpu.sync_copy(zeros_hbm, hist_v)

    lane_base = lax.broadcasted_iota(jnp.int32, (16,), 0) * K
    ones16 = jnp.ones((16,), jnp.float32)
    for t in range(RPW // 16):
        iv = idx_v[t // 8, pl.ds((t % 8) * 16, 16)]
        plsc.addupdate_scatter(hist_v, [lane_base + iv], ones16)

    def _reduce(c, carry):
        acc = hist_v[pl.ds(c * 16, 16)]
        for lane in range(1, 16):
            acc = acc + hist_v[pl.ds(lane * K + c * 16, 16)]
        cnt_v[pl.ds(c * 16, 16)] = acc
        return carry

    lax.fori_loop(0, K // 16, _reduce, 0)
    pltpu.sync_copy(cnt_v, cnt_hbm.at[pl.ds(wid * K, K)])

    for c in copies:
        c.wait()
    pltpu.sync_copy(rows_v, zq_hbm.at[pl.ds(wid * RPW, RPW)])


_sc_kernel = functools.partial(
    pl.kernel,
    mesh=plsc.VectorSubcoreMesh(
        core_axis_name="c", subcore_axis_name="s",
        num_cores=2, num_subcores=16),
    out_type=[
        jax.ShapeDtypeStruct((ROWS, DIM), jnp.float32),
        jax.ShapeDtypeStruct((NW * K,), jnp.float32),
    ],
    scratch_types=[
        pltpu.VMEM((NCH, CHUNK), jnp.int32),
        pltpu.VMEM((RPW, DIM), jnp.float32),
        pltpu.VMEM((16 * K,), jnp.float32),
        pltpu.VMEM((K,), jnp.float32),
        pltpu.SemaphoreType.DMA,
    ],
    compiler_params=pltpu.CompilerParams(
        needs_layout_passes=False, use_tc_tiling_on_sc=False),
)(_sc_gather_hist)


def kernel(z, codebook):
    z_flat = z.reshape(ROWS, DIM)
    zt = jnp.transpose(z, (0, 2, 1)).reshape(NBLK * DIM, SEQ)
    z2 = jnp.sum(z_flat ** 2, axis=1).reshape(NBLK, 1, SEQ)
    c2 = jnp.sum(codebook ** 2, axis=1)
    c2b = jnp.broadcast_to(c2[:, None], (K, SEQ))
    idx3, loss = pl.pallas_call(
        _dist_argmin,
        grid=(NBLK,),
        in_specs=[
            pl.BlockSpec((DIM, SEQ), lambda i: (i, 0)),
            pl.BlockSpec((K, DIM), lambda i: (0, 0)),
            pl.BlockSpec((K, SEQ), lambda i: (0, 0)),
            pl.BlockSpec((1, 1, SEQ), lambda i: (i, 0, 0)),
        ],
        out_specs=[
            pl.BlockSpec((1, 1, SEQ), lambda i: (i, 0, 0)),
            pl.BlockSpec((1, 1), lambda i: (0, 0)),
        ],
        out_shape=[
            jax.ShapeDtypeStruct((NBLK, 1, SEQ), jnp.int32),
            jax.ShapeDtypeStruct((1, 1), jnp.float32),
        ],
    )(zt, codebook + codebook, c2b, z2)
    indices = idx3.reshape(z.shape[:-1])
    idx_w = idx3.reshape(NW, NCH, CHUNK)
    zq, counts = _sc_kernel(codebook, idx_w, jnp.zeros((16 * K,), jnp.float32))
    z_q_st = zq.reshape(z.shape)
    loss_scalar = loss[0, 0] / jnp.float32(ROWS * DIM)
    usage = jnp.sum(counts.reshape(NW, K), axis=0) / jnp.float32(ROWS)
    return (z_q_st, indices, loss_scalar, loss_scalar, usage)


# speedup vs baseline: 1.1177x; 1.1177x over previous
import functools

import jax
import jax.numpy as jnp
from jax import lax
from jax.experimental import pallas as pl
from jax.experimental.pallas import tpu as pltpu
from jax.experimental.pallas import tpu_sc as plsc

K = 1024
DIM = 64
ROWS = 64 * 576
BLK = 2048
NBLK = ROWS // BLK

NW = 32
RPW = ROWS // NW
CHUNK = 128
NCH = RPW // CHUNK
IDX_ROWS = ROWS // CHUNK


def _dist_argmin(z_ref, cb_ref, cb2_ref, idx_ref, loss_ref):
    i = pl.program_id(0)
    zb = z_ref[...]
    cb = cb_ref[...]
    logits2 = lax.dot_general(
        cb2_ref[...], zb, (((1,), (1,)), ((), ())),
        preferred_element_type=jnp.float32)
    z2 = jnp.sum(zb * zb, axis=1)
    c2 = jnp.sum(cb * cb, axis=1)
    score = (c2[:, None] + z2[None, :]) - logits2
    colmin = jnp.min(score, axis=0, keepdims=True)
    iota_f = lax.broadcasted_iota(
        jnp.int32, score.shape, 0).astype(jnp.float32)
    idx_f = jnp.min(jnp.where(score == colmin, iota_f, jnp.float32(K)),
                    axis=0)
    idx_ref[0, 0, :] = idx_f.astype(jnp.int32)
    loss_part = jnp.sum(colmin).reshape(1, 1)

    @pl.when(i == 0)
    def _init():
        loss_ref[...] = loss_part

    @pl.when(i > 0)
    def _acc():
        loss_ref[...] += loss_part


def _sc_gather_hist(cb_hbm, idx_hbm, zeros_hbm, zq_hbm, cnt_hbm,
                    idx_v, rows_v, hist_v, cnt_v, sem):
    wid = lax.axis_index("s") * 2 + lax.axis_index("c")

    pltpu.sync_copy(idx_hbm.at[wid], idx_v)
    copies = [
        pltpu.async_copy(cb_hbm.at[idx_v.at[j]],
                         rows_v.at[pl.ds(j * CHUNK, CHUNK)], sem)
        for j in range(NCH)
    ]

    pltpu.sync_copy(zeros_hbm, hist_v)

    lane_base = lax.broadcasted_iota(jnp.int32, (16,), 0) * K
    ones16 = jnp.ones((16,), jnp.float32)
    for t in range(RPW // 16):
        iv = idx_v[t // 8, pl.ds((t % 8) * 16, 16)]
        plsc.addupdate_scatter(hist_v, [lane_base + iv], ones16)

    def _reduce(c, carry):
        acc = hist_v[pl.ds(c * 16, 16)]
        for lane in range(1, 16):
            acc = acc + hist_v[pl.ds(lane * K + c * 16, 16)]
        cnt_v[pl.ds(c * 16, 16)] = acc
        return carry

    lax.fori_loop(0, K // 16, _reduce, 0)
    pltpu.sync_copy(cnt_v, cnt_hbm.at[pl.ds(wid * K, K)])

    for c in copies:
        c.wait()
    pltpu.sync_copy(rows_v, zq_hbm.at[pl.ds(wid * RPW, RPW)])


_sc_kernel = functools.partial(
    pl.kernel,
    mesh=plsc.VectorSubcoreMesh(
        core_axis_name="c", subcore_axis_name="s",
        num_cores=2, num_subcores=16),
    out_type=[
        jax.ShapeDtypeStruct((ROWS, DIM), jnp.float32),
        jax.ShapeDtypeStruct((NW * K,), jnp.float32),
    ],
    scratch_types=[
        pltpu.VMEM((NCH, CHUNK), jnp.int32),
        pltpu.VMEM((RPW, DIM), jnp.float32),
        pltpu.VMEM((16 * K,), jnp.float32),
        pltpu.VMEM((K,), jnp.float32),
        pltpu.SemaphoreType.DMA,
    ],
    compiler_params=pltpu.CompilerParams(
        needs_layout_passes=False, use_tc_tiling_on_sc=False),
)(_sc_gather_hist)


def kernel(z, codebook):
    z_flat = z.reshape(ROWS, DIM)
    idx3, loss = pl.pallas_call(
        _dist_argmin,
        grid=(NBLK,),
        in_specs=[
            pl.BlockSpec((BLK, DIM), lambda i: (i, 0)),
            pl.BlockSpec((K, DIM), lambda i: (0, 0)),
            pl.BlockSpec((K, DIM), lambda i: (0, 0)),
        ],
        out_specs=[
            pl.BlockSpec((1, 1, BLK), lambda i: (i, 0, 0)),
            pl.BlockSpec((1, 1), lambda i: (0, 0)),
        ],
        out_shape=[
            jax.ShapeDtypeStruct((NBLK, 1, BLK), jnp.int32),
            jax.ShapeDtypeStruct((1, 1), jnp.float32),
        ],
    )(z_flat, codebook, codebook + codebook)
    indices = idx3.reshape(z.shape[:-1])
    idx_w = idx3.reshape(NW, NCH, CHUNK)
    zq, counts = _sc_kernel(codebook, idx_w, jnp.zeros((16 * K,), jnp.float32))
    z_q_st = zq.reshape(z.shape)
    loss_scalar = loss[0, 0] / jnp.float32(ROWS * DIM)
    usage = jnp.sum(counts.reshape(NW, K), axis=0) / jnp.float32(ROWS)
    return (z_q_st, indices, loss_scalar, loss_scalar, usage)


# speedup vs baseline: 1.1249x; 1.0064x over previous
import functools

import jax
import jax.numpy as jnp
from jax import lax
from jax.experimental import pallas as pl
from jax.experimental.pallas import tpu as pltpu
from jax.experimental.pallas import tpu_sc as plsc

K = 1024
DIM = 64
ROWS = 64 * 576
BLK = 3072
NBLK = ROWS // BLK

NW = 32
RPW = ROWS // NW
CHUNK = 128
NCH = RPW // CHUNK
IDX_ROWS = ROWS // CHUNK


def _dist_argmin(z_ref, cb_ref, cb2_ref, idx_ref, loss_ref):
    i = pl.program_id(0)
    zb = z_ref[...]
    cb = cb_ref[...]
    logits2 = lax.dot_general(
        cb2_ref[...], zb, (((1,), (1,)), ((), ())),
        preferred_element_type=jnp.float32)
    z2 = jnp.sum(zb * zb, axis=1)
    c2 = jnp.sum(cb * cb, axis=1)
    score = (c2[:, None] + z2[None, :]) - logits2
    colmin = jnp.min(score, axis=0, keepdims=True)
    iota_f = lax.broadcasted_iota(
        jnp.int32, score.shape, 0).astype(jnp.float32)
    idx_f = jnp.min(jnp.where(score == colmin, iota_f, jnp.float32(K)),
                    axis=0)
    idx_ref[0, 0, :] = idx_f.astype(jnp.int32)
    loss_part = jnp.sum(colmin).reshape(1, 1)

    @pl.when(i == 0)
    def _init():
        loss_ref[...] = loss_part

    @pl.when(i > 0)
    def _acc():
        loss_ref[...] += loss_part


def _sc_gather_hist(cb_hbm, idx_hbm, zeros_hbm, zq_hbm, cnt_hbm,
                    idx_v, rows_v, hist_v, cnt_v, sem):
    wid = lax.axis_index("s") * 2 + lax.axis_index("c")

    pltpu.sync_copy(idx_hbm.at[wid], idx_v)
    copies = [
        pltpu.async_copy(cb_hbm.at[idx_v.at[j]],
                         rows_v.at[pl.ds(j * CHUNK, CHUNK)], sem)
        for j in range(NCH)
    ]

    pltpu.sync_copy(zeros_hbm, hist_v)

    lane_base = lax.broadcasted_iota(jnp.int32, (16,), 0) * K
    ones16 = jnp.ones((16,), jnp.float32)
    for t in range(RPW // 16):
        iv = idx_v[t // 8, pl.ds((t % 8) * 16, 16)]
        plsc.addupdate_scatter(hist_v, [lane_base + iv], ones16)

    def _reduce(c, carry):
        acc = hist_v[pl.ds(c * 16, 16)]
        for lane in range(1, 16):
            acc = acc + hist_v[pl.ds(lane * K + c * 16, 16)]
        cnt_v[pl.ds(c * 16, 16)] = acc
        return carry

    lax.fori_loop(0, K // 16, _reduce, 0)
    pltpu.sync_copy(cnt_v, cnt_hbm.at[pl.ds(wid * K, K)])

    for c in copies:
        c.wait()
    pltpu.sync_copy(rows_v, zq_hbm.at[pl.ds(wid * RPW, RPW)])


_sc_kernel = functools.partial(
    pl.kernel,
    mesh=plsc.VectorSubcoreMesh(
        core_axis_name="c", subcore_axis_name="s",
        num_cores=2, num_subcores=16),
    out_type=[
        jax.ShapeDtypeStruct((ROWS, DIM), jnp.float32),
        jax.ShapeDtypeStruct((NW * K,), jnp.float32),
    ],
    scratch_types=[
        pltpu.VMEM((NCH, CHUNK), jnp.int32),
        pltpu.VMEM((RPW, DIM), jnp.float32),
        pltpu.VMEM((16 * K,), jnp.float32),
        pltpu.VMEM((K,), jnp.float32),
        pltpu.SemaphoreType.DMA,
    ],
    compiler_params=pltpu.CompilerParams(
        needs_layout_passes=False, use_tc_tiling_on_sc=False),
)(_sc_gather_hist)


def kernel(z, codebook):
    z_flat = z.reshape(ROWS, DIM)
    idx3, loss = pl.pallas_call(
        _dist_argmin,
        grid=(NBLK,),
        in_specs=[
            pl.BlockSpec((BLK, DIM), lambda i: (i, 0)),
            pl.BlockSpec((K, DIM), lambda i: (0, 0)),
            pl.BlockSpec((K, DIM), lambda i: (0, 0)),
        ],
        out_specs=[
            pl.BlockSpec((1, 1, BLK), lambda i: (i, 0, 0)),
            pl.BlockSpec((1, 1), lambda i: (0, 0)),
        ],
        out_shape=[
            jax.ShapeDtypeStruct((NBLK, 1, BLK), jnp.int32),
            jax.ShapeDtypeStruct((1, 1), jnp.float32),
        ],
    )(z_flat, codebook, codebook + codebook)
    indices = idx3.reshape(z.shape[:-1])
    idx_w = idx3.reshape(NW, NCH, CHUNK)
    zq, counts = _sc_kernel(codebook, idx_w, jnp.zeros((16 * K,), jnp.float32))
    z_q_st = zq.reshape(z.shape)
    loss_scalar = loss[0, 0] / jnp.float32(ROWS * DIM)
    usage = jnp.sum(counts.reshape(NW, K), axis=0) / jnp.float32(ROWS)
    return (z_q_st, indices, loss_scalar, loss_scalar, usage)
